# baseline (speedup 1.0000x reference)
import functools

import jax
import jax.numpy as jnp
from jax import lax
from jax.experimental import pallas as pl
from jax.experimental.pallas import tpu as pltpu
from jax.experimental.pallas import tpu_sc as plsc

_N = 10000
_E = 320000
_G = 64
_NW = 32
_RPW = 320
_NPAD = _NW * _RPW
_SENTINEL = 1 << 29


def _elu(x):
    return jnp.where(x > 0, x, jnp.exp(jnp.minimum(x, 0.0)) - 1.0)


def _mlp3_body(x_ref, w1, b1, w2, b2, w3, b3, o_ref, *, elu_out):
    h = x_ref[...]
    h = _elu(jnp.dot(h, w1[...], preferred_element_type=jnp.float32) + b1[...])
    h = _elu(jnp.dot(h, w2[...], preferred_element_type=jnp.float32) + b2[...])
    h = jnp.dot(h, w3[...], preferred_element_type=jnp.float32) + b3[...]
    if elu_out:
        h = _elu(h)
    o_ref[...] = h


def _mlp3(x, params, elu_out=False, block=2048, pad_to=None, trim=True):
    (w1, b1), (w2, b2), (w3, b3) = params
    m, k = x.shape
    out_dim = w3.shape[1]
    mp = pad_to or ((m + block - 1) // block) * block
    assert mp % block == 0
    if mp != m:
        x = jnp.pad(x, ((0, mp - m), (0, 0)))
    grid = mp // block
    full = lambda r, c: pl.BlockSpec((r, c), lambda i: (0, 0))
    out = pl.pallas_call(
        functools.partial(_mlp3_body, elu_out=elu_out),
        grid=(grid,),
        in_specs=[
            pl.BlockSpec((block, k), lambda i: (i, 0)),
            full(*w1.shape), full(1, b1.shape[0]),
            full(*w2.shape), full(1, b2.shape[0]),
            full(*w3.shape), full(1, b3.shape[0]),
        ],
        out_specs=pl.BlockSpec((block, out_dim), lambda i: (i, 0)),
        out_shape=jax.ShapeDtypeStruct((mp, out_dim), jnp.float32),
    )(x, w1, b1.reshape(1, -1), w2, b2.reshape(1, -1), w3, b3.reshape(1, -1))
    return out[:m] if trim else out


def _mlp3_pair_body(xa_ref, xb_ref, w1a, w1b, b1, w2, b2, w3, b3, o_ref):
    h = (jnp.dot(xa_ref[...], w1a[...], preferred_element_type=jnp.float32)
         + jnp.dot(xb_ref[...], w1b[...], preferred_element_type=jnp.float32)
         + b1[...])
    h = _elu(h)
    h = _elu(jnp.dot(h, w2[...], preferred_element_type=jnp.float32) + b2[...])
    h = jnp.dot(h, w3[...], preferred_element_type=jnp.float32) + b3[...]
    o_ref[...] = h.astype(o_ref.dtype)


def _mlp3_pair(xa, xb, w1a, w1b, b1, p2, p3, block=2048):
    (w2, b2), (w3, b3) = p2, p3
    m = xa.shape[0]
    out_dim = w3.shape[1]
    assert m % block == 0 and xb.shape[0] == m
    full = lambda r, c: pl.BlockSpec((r, c), lambda i: (0, 0))
    return pl.pallas_call(
        _mlp3_pair_body,
        grid=(m // block,),
        in_specs=[
            pl.BlockSpec((block, xa.shape[1]), lambda i: (i, 0)),
            pl.BlockSpec((block, xb.shape[1]), lambda i: (i, 0)),
            full(*w1a.shape), full(*w1b.shape), full(1, b1.shape[0]),
            full(*w2.shape), full(1, b2.shape[0]),
            full(*w3.shape), full(1, b3.shape[0]),
        ],
        out_specs=pl.BlockSpec((block, out_dim), lambda i: (i, 0)),
        out_shape=jax.ShapeDtypeStruct((m, out_dim), jnp.bfloat16),
    )(xa, xb, w1a, w1b, b1.reshape(1, -1), w2, b2.reshape(1, -1),
      w3, b3.reshape(1, -1))


def _gather_sc(table, idx):
    v, d = table.shape
    mp = idx.shape[0]
    per_w = mp // _NW
    assert per_w * _NW == mp and (d * 4) % 64 == 0
    kb = 128 if d > 64 else 256
    nk = per_w // kb
    assert nk * kb == per_w
    mesh = plsc.VectorSubcoreMesh(core_axis_name="c", subcore_axis_name="s")

    @functools.partial(
        pl.kernel,
        mesh=mesh,
        compiler_params=pltpu.CompilerParams(
            needs_layout_passes=False, use_tc_tiling_on_sc=False),
        out_type=jax.ShapeDtypeStruct((mp, d), jnp.float32),
        scratch_types=[
            pltpu.VMEM((per_w,), jnp.int32),
            pltpu.VMEM((2 * kb, d), jnp.float32),
            pltpu.SemaphoreType.DMA,
        ],
    )
    def k(table_hbm, idx_hbm, out_hbm, idxv, buf, sem):
        wid = lax.axis_index("c") * 16 + lax.axis_index("s")
        wbase = wid * per_w
        pltpu.sync_copy(idx_hbm.at[pl.ds(wbase, per_w)], idxv)

        def _start(i):
            pltpu.make_async_copy(
                table_hbm.at[idxv.at[pl.ds(i * kb, kb)]],
                buf.at[pl.ds((i % 2) * kb, kb)], sem).start()

        def _drain():
            pltpu.make_async_copy(
                table_hbm.at[idxv.at[pl.ds(0, kb)]],
                buf.at[pl.ds(0, kb)], sem).wait()

        _start(jnp.int32(0))

        def body(i, _):
            _start(jnp.minimum(i + 1, nk - 1))
            _drain()
            pltpu.sync_copy(buf.at[pl.ds((i % 2) * kb, kb)],
                            out_hbm.at[pl.ds(wbase + i * kb, kb)])
            return 0

        lax.fori_loop(0, nk, body, 0)
        _drain()

    return k(table, idx)


def _segmax_sc(msg, dst):
    mp, d = msg.shape
    ch = 8192
    fb = 512
    rb = 256
    sb = 32
    assert mp % ch == 0 and d % 16 == 0
    n_chunks = mp // ch
    ncol = d // 16
    mesh = plsc.VectorSubcoreMesh(core_axis_name="c", subcore_axis_name="s")

    @functools.partial(
        pl.kernel,
        mesh=mesh,
        compiler_params=pltpu.CompilerParams(needs_layout_passes=False),
        out_type=jax.ShapeDtypeStruct((_NPAD, d), jnp.int32),
        scratch_types=[
            pltpu.VMEM((_RPW + 1, d), jnp.int32),
            pltpu.VMEM((ch,), jnp.int32),
            pltpu.VMEM((fb,), jnp.int32),
            pltpu.VMEM((fb,), jnp.int32),
            pltpu.VMEM((rb, d), jnp.int32),
            pltpu.SemaphoreType.DMA,
        ],
    )
    def k(msg_hbm, dst_hbm, out_hbm, acc, dstv, fid, rid, rows, sem):
        wid = lax.axis_index("c") * 16 + lax.axis_index("s")
        base = wid * _RPW
        lanes = lax.iota(jnp.int32, 16)
        neg = jnp.full((16,), -8323200, jnp.int32)

        def init_row(r, _):
            for kk in range(ncol):
                acc[r, pl.ds(kk * 16, 16)] = neg
            return 0
        lax.fori_loop(0, _RPW + 1, init_row, 0)

        def reset_bufs():
            for t in range(fb // 16):
                fid[pl.ds(t * 16, 16)] = t * 16 + lanes
                rid[pl.ds(t * 16, 16)] = jnp.full((16,), _RPW, jnp.int32)
        reset_bufs()

        nslot = rb // sb

        def _start(i):
            pltpu.make_async_copy(
                msg_hbm.at[fid.at[pl.ds(i * sb, sb)]],
                rows.at[pl.ds((i % nslot) * sb, sb)], sem).start()

        def _drain():
            pltpu.make_async_copy(
                msg_hbm.at[fid.at[pl.ds(0, sb)]],
                rows.at[pl.ds(0, sb)], sem).wait()

        def flush(cnt):
            nsb_d = jnp.maximum((cnt + sb - 1) // sb, 1)
            _start(jnp.int32(0))

            def body(i, _):
                _drain()
                _start(jnp.minimum(i + 1, nsb_d - 1))
                slot = (i % nslot) * sb

                def acc_row(j16, _):
                    rv = rid[pl.ds(i * sb + j16 * 16, 16)]
                    for l in range(16):
                        r = rv[l]
                        j = slot + j16 * 16 + l
                        for kk in range(ncol):
                            sl = pl.ds(kk * 16, 16)
                            a = plsc.bitcast(acc[r, sl], jnp.bfloat16)
                            b = plsc.bitcast(rows[j, sl], jnp.bfloat16)
                            acc[r, sl] = plsc.bitcast(jnp.maximum(a, b), jnp.int32)
                    return 0
                lax.fori_loop(0, sb // 16, acc_row, 0)
                return 0

            lax.fori_loop(0, nsb_d, body, 0)
            _drain()
            reset_bufs()
            return jnp.int32(0)

        def chunk(c, cnt):
            pltpu.sync_copy(dst_hbm.at[pl.ds(c * ch, ch)], dstv)

            def block8(b, cnt):
                cnt = lax.cond(cnt > fb - 128, flush, lambda x: x, cnt)
                for g8 in range(8):
                    g = b * 8 + g8
                    v = dstv[pl.ds(g * 16, 16)]
                    rel = v - base
                    m = plsc.bitcast(rel, jnp.uint32) < jnp.uint32(_RPW)
                    eid = c * ch + g * 16 + lanes
                    plsc.store_compressed(fid.at[pl.ds(cnt, 16)], eid, mask=m)
                    plsc.store_compressed(rid.at[pl.ds(cnt, 16)], rel, mask=m)
                    cnt = cnt + plsc.all_reduce_population_count(m)[0]
                return cnt

            return lax.fori_loop(0, ch // 128, block8, cnt)

        cnt = lax.fori_loop(0, n_chunks, chunk, jnp.int32(0))
        flush(cnt)
        pltpu.sync_copy(acc.at[pl.ds(0, _RPW)], out_hbm.at[pl.ds(base, _RPW)])

    return k(msg, dst)


_MP = 335872


def _split_w1(w1, nf, da):
    wf, wp = w1[:nf], w1[nf:]
    w1a = jnp.zeros((da, w1.shape[1]), jnp.float32)
    w1a = w1a.at[:nf].set(wf).at[nf:nf + 3].set(wp)
    w1b = jnp.zeros((16, w1.shape[1]), jnp.float32).at[:3].set(-wp)
    return w1a, w1b


def _conv_layer(feat_pos_tab, gpd, src_pad, dst_pad, nf, local_p, global_p):
    (w1, b1), p2, p3 = local_p
    gsrc = _gather_sc(feat_pos_tab, src_pad)
    w1a, w1b = _split_w1(w1, nf, feat_pos_tab.shape[1])
    msg = _mlp3_pair(gsrc, gpd, w1a, w1b, b1, p2, p3)
    msg_u32 = lax.bitcast_convert_type(
        msg.reshape(_MP, msg.shape[1] // 2, 2), jnp.uint32).astype(jnp.int32)
    agg = lax.bitcast_convert_type(
        _segmax_sc(msg_u32, dst_pad), jnp.bfloat16).reshape(_NPAD, -1)
    agg = agg[:_N].astype(jnp.float32)
    return _mlp3(agg, global_p, elu_out=True)


def kernel(x, pos, params, edge_index, batch):
    loop = jnp.arange(_N, dtype=edge_index.dtype)
    pad_ids = (jnp.arange(_MP - _E - _N, dtype=jnp.int32) * 7) % _N
    src_pad = jnp.concatenate([edge_index[0], loop, pad_ids])
    dst_safe = jnp.concatenate([edge_index[1], loop, pad_ids])
    dst_pad = jnp.concatenate([
        edge_index[1], loop,
        jnp.full((_MP - _E - _N,), _SENTINEL, edge_index.dtype),
    ])

    posp = jnp.pad(pos, ((0, 0), (0, 13)))
    gpd = _gather_sc(posp, dst_safe)
    t1 = jnp.pad(jnp.concatenate([x, pos], axis=1), ((0, 0), (0, 10)))
    x1 = _conv_layer(t1, gpd, src_pad, dst_pad, 3, params['ln1'], params['gn1'])
    t2 = jnp.pad(jnp.concatenate([x1, pos], axis=1), ((0, 0), (0, 13)))
    x2 = _conv_layer(t2, gpd, src_pad, dst_pad, 128, params['ln2'], params['gn2'])

    x_add = jax.ops.segment_sum(x2, batch, num_segments=_G)
    cnt = jax.ops.segment_sum(jnp.ones((_N, 1), jnp.float32), batch, num_segments=_G)
    x_mean = x_add / jnp.maximum(cnt, 1.0)
    x_max = jax.ops.segment_max(x2, batch, num_segments=_G)
    h = jnp.concatenate([x_max, x_mean, x_add], axis=1)
    wl, bl = params['lin1']
    return h @ wl + bl


# speedup vs baseline: 2.1551x; 2.1551x over previous
import functools

import jax
import jax.numpy as jnp
from jax import lax
from jax.experimental import pallas as pl
from jax.experimental.pallas import tpu as pltpu
from jax.experimental.pallas import tpu_sc as plsc

_N = 10000
_E = 320000
_G = 64
_NW = 32
_RPW = 320
_NPAD = _NW * _RPW
_SENTINEL = 1 << 29


def _elu(x):
    return jnp.where(x > 0, x, jnp.exp(jnp.minimum(x, 0.0)) - 1.0)


def _mlp3_body(x_ref, w1, b1, w2, b2, w3, b3, o_ref, *, elu_out):
    h = x_ref[...]
    h = _elu(jnp.dot(h, w1[...], preferred_element_type=jnp.float32) + b1[...])
    h = _elu(jnp.dot(h, w2[...], preferred_element_type=jnp.float32) + b2[...])
    h = jnp.dot(h, w3[...], preferred_element_type=jnp.float32) + b3[...]
    if elu_out:
        h = _elu(h)
    o_ref[...] = h


def _mlp3(x, params, elu_out=False, block=2048, pad_to=None, trim=True):
    (w1, b1), (w2, b2), (w3, b3) = params
    m, k = x.shape
    out_dim = w3.shape[1]
    mp = pad_to or ((m + block - 1) // block) * block
    assert mp % block == 0
    if mp != m:
        x = jnp.pad(x, ((0, mp - m), (0, 0)))
    grid = mp // block
    full = lambda r, c: pl.BlockSpec((r, c), lambda i: (0, 0))
    out = pl.pallas_call(
        functools.partial(_mlp3_body, elu_out=elu_out),
        grid=(grid,),
        in_specs=[
            pl.BlockSpec((block, k), lambda i: (i, 0)),
            full(*w1.shape), full(1, b1.shape[0]),
            full(*w2.shape), full(1, b2.shape[0]),
            full(*w3.shape), full(1, b3.shape[0]),
        ],
        out_specs=pl.BlockSpec((block, out_dim), lambda i: (i, 0)),
        out_shape=jax.ShapeDtypeStruct((mp, out_dim), jnp.float32),
    )(x, w1, b1.reshape(1, -1), w2, b2.reshape(1, -1), w3, b3.reshape(1, -1))
    return out[:m] if trim else out


def _mlp3_pair_body(xa_ref, xb_ref, w1a, w1b, b1, w2, b2, w3, b3, o_ref):
    h = (jnp.dot(xa_ref[...], w1a[...], preferred_element_type=jnp.float32)
         + jnp.dot(xb_ref[...], w1b[...], preferred_element_type=jnp.float32)
         + b1[...])
    h = _elu(h)
    h = _elu(jnp.dot(h, w2[...], preferred_element_type=jnp.float32) + b2[...])
    h = jnp.dot(h, w3[...], preferred_element_type=jnp.float32) + b3[...]
    half = h.shape[1] // 2
    lo = pltpu.bitcast(h[:, :half].astype(jnp.bfloat16), jnp.uint16).astype(jnp.uint32)
    hi = pltpu.bitcast(h[:, half:].astype(jnp.bfloat16), jnp.uint16).astype(jnp.uint32)
    o_ref[...] = pltpu.bitcast(lo | (hi << 16), jnp.int32)


def _mlp3_pair(xa, xb, w1a, w1b, b1, p2, p3, block=2048):
    (w2, b2), (w3, b3) = p2, p3
    m = xa.shape[0]
    out_dim = w3.shape[1]
    assert m % block == 0 and xb.shape[0] == m
    full = lambda r, c: pl.BlockSpec((r, c), lambda i: (0, 0))
    return pl.pallas_call(
        _mlp3_pair_body,
        grid=(m // block,),
        in_specs=[
            pl.BlockSpec((block, xa.shape[1]), lambda i: (i, 0)),
            pl.BlockSpec((block, xb.shape[1]), lambda i: (i, 0)),
            full(*w1a.shape), full(*w1b.shape), full(1, b1.shape[0]),
            full(*w2.shape), full(1, b2.shape[0]),
            full(*w3.shape), full(1, b3.shape[0]),
        ],
        out_specs=pl.BlockSpec((block, out_dim // 2), lambda i: (i, 0)),
        out_shape=jax.ShapeDtypeStruct((m, out_dim // 2), jnp.int32),
    )(xa, xb, w1a, w1b, b1.reshape(1, -1), w2, b2.reshape(1, -1),
      w3, b3.reshape(1, -1))


def _gather_sc(table, idx):
    v, d = table.shape
    mp = idx.shape[0]
    per_w = mp // _NW
    assert per_w * _NW == mp and (d * 4) % 64 == 0
    kb = 128 if d > 64 else 256
    nk = per_w // kb
    assert nk * kb == per_w
    mesh = plsc.VectorSubcoreMesh(core_axis_name="c", subcore_axis_name="s")

    @functools.partial(
        pl.kernel,
        mesh=mesh,
        compiler_params=pltpu.CompilerParams(
            needs_layout_passes=False, use_tc_tiling_on_sc=False),
        out_type=jax.ShapeDtypeStruct((mp, d), jnp.float32),
        scratch_types=[
            pltpu.VMEM((per_w,), jnp.int32),
            pltpu.VMEM((2 * kb, d), jnp.float32),
            pltpu.SemaphoreType.DMA,
        ],
    )
    def k(table_hbm, idx_hbm, out_hbm, idxv, buf, sem):
        wid = lax.axis_index("c") * 16 + lax.axis_index("s")
        wbase = wid * per_w
        pltpu.sync_copy(idx_hbm.at[pl.ds(wbase, per_w)], idxv)

        def _start(i):
            pltpu.make_async_copy(
                table_hbm.at[idxv.at[pl.ds(i * kb, kb)]],
                buf.at[pl.ds((i % 2) * kb, kb)], sem).start()

        def _drain():
            pltpu.make_async_copy(
                table_hbm.at[idxv.at[pl.ds(0, kb)]],
                buf.at[pl.ds(0, kb)], sem).wait()

        _start(jnp.int32(0))

        def body(i, _):
            _start(jnp.minimum(i + 1, nk - 1))
            _drain()
            pltpu.sync_copy(buf.at[pl.ds((i % 2) * kb, kb)],
                            out_hbm.at[pl.ds(wbase + i * kb, kb)])
            return 0

        lax.fori_loop(0, nk, body, 0)
        _drain()

    return k(table, idx)


def _segmax_sc(msg, dst):
    mp, d = msg.shape
    ch = 8192
    fb = 512
    rb = 256
    sb = 32
    assert mp % ch == 0 and d % 16 == 0
    n_chunks = mp // ch
    ncol = d // 16
    mesh = plsc.VectorSubcoreMesh(core_axis_name="c", subcore_axis_name="s")

    @functools.partial(
        pl.kernel,
        mesh=mesh,
        compiler_params=pltpu.CompilerParams(needs_layout_passes=False),
        out_type=jax.ShapeDtypeStruct((_NPAD, d), jnp.int32),
        scratch_types=[
            pltpu.VMEM((_RPW + 1, d), jnp.int32),
            pltpu.VMEM((ch,), jnp.int32),
            pltpu.VMEM((fb,), jnp.int32),
            pltpu.VMEM((fb,), jnp.int32),
            pltpu.VMEM((rb, d), jnp.int32),
            pltpu.SemaphoreType.DMA,
        ],
    )
    def k(msg_hbm, dst_hbm, out_hbm, acc, dstv, fid, rid, rows, sem):
        wid = lax.axis_index("c") * 16 + lax.axis_index("s")
        base = wid * _RPW
        lanes = lax.iota(jnp.int32, 16)
        neg = jnp.full((16,), -8323200, jnp.int32)

        def init_row(r, _):
            for kk in range(ncol):
                acc[r, pl.ds(kk * 16, 16)] = neg
            return 0
        lax.fori_loop(0, _RPW + 1, init_row, 0)

        def reset_bufs():
            for t in range(fb // 16):
                fid[pl.ds(t * 16, 16)] = t * 16 + lanes
                rid[pl.ds(t * 16, 16)] = jnp.full((16,), _RPW, jnp.int32)
        reset_bufs()

        nslot = rb // sb

        def _start(i):
            pltpu.make_async_copy(
                msg_hbm.at[fid.at[pl.ds(i * sb, sb)]],
                rows.at[pl.ds((i % nslot) * sb, sb)], sem).start()

        def _drain():
            pltpu.make_async_copy(
                msg_hbm.at[fid.at[pl.ds(0, sb)]],
                rows.at[pl.ds(0, sb)], sem).wait()

        def flush(cnt):
            nsb_d = jnp.maximum((cnt + sb - 1) // sb, 1)
            _start(jnp.int32(0))

            def body(i, _):
                _drain()
                _start(jnp.minimum(i + 1, nsb_d - 1))
                slot = (i % nslot) * sb

                def acc_row(j16, _):
                    rv = rid[pl.ds(i * sb + j16 * 16, 16)]
                    for l in range(16):
                        r = rv[l]
                        j = slot + j16 * 16 + l
                        for kk in range(ncol):
                            sl = pl.ds(kk * 16, 16)
                            a = plsc.bitcast(acc[r, sl], jnp.bfloat16)
                            b = plsc.bitcast(rows[j, sl], jnp.bfloat16)
                            acc[r, sl] = plsc.bitcast(jnp.maximum(a, b), jnp.int32)
                    return 0
                lax.fori_loop(0, sb // 16, acc_row, 0)
                return 0

            lax.fori_loop(0, nsb_d, body, 0)
            _drain()
            reset_bufs()
            return jnp.int32(0)

        def chunk(c, cnt):
            pltpu.sync_copy(dst_hbm.at[pl.ds(c * ch, ch)], dstv)

            def block8(b, cnt):
                cnt = lax.cond(cnt > fb - 128, flush, lambda x: x, cnt)
                for g8 in range(8):
                    g = b * 8 + g8
                    v = dstv[pl.ds(g * 16, 16)]
                    rel = v - base
                    m = plsc.bitcast(rel, jnp.uint32) < jnp.uint32(_RPW)
                    eid = c * ch + g * 16 + lanes
                    plsc.store_compressed(fid.at[pl.ds(cnt, 16)], eid, mask=m)
                    plsc.store_compressed(rid.at[pl.ds(cnt, 16)], rel, mask=m)
                    cnt = cnt + plsc.all_reduce_population_count(m)[0]
                return cnt

            return lax.fori_loop(0, ch // 128, block8, cnt)

        cnt = lax.fori_loop(0, n_chunks, chunk, jnp.int32(0))
        flush(cnt)
        pltpu.sync_copy(acc.at[pl.ds(0, _RPW)], out_hbm.at[pl.ds(base, _RPW)])

    return k(msg, dst)


_MP = 335872


def _split_w1(w1, nf, da):
    wf, wp = w1[:nf], w1[nf:]
    w1a = jnp.zeros((da, w1.shape[1]), jnp.float32)
    w1a = w1a.at[:nf].set(wf).at[nf:nf + 3].set(wp)
    w1b = jnp.zeros((16, w1.shape[1]), jnp.float32).at[:3].set(-wp)
    return w1a, w1b


def _conv_layer(feat_pos_tab, gpd, src_pad, dst_pad, nf, local_p, global_p):
    (w1, b1), p2, p3 = local_p
    gsrc = _gather_sc(feat_pos_tab, src_pad)
    w1a, w1b = _split_w1(w1, nf, feat_pos_tab.shape[1])
    msg = _mlp3_pair(gsrc, gpd, w1a, w1b, b1, p2, p3)
    agg32 = jnp.uint32(_segmax_sc(msg, dst_pad)[:_N])
    lo = lax.bitcast_convert_type((agg32 & 0xFFFF).astype(jnp.uint16), jnp.bfloat16)
    hi = lax.bitcast_convert_type((agg32 >> 16).astype(jnp.uint16), jnp.bfloat16)
    agg = jnp.concatenate([lo, hi], axis=1).astype(jnp.float32)
    return _mlp3(agg, global_p, elu_out=True)


def kernel(x, pos, params, edge_index, batch):
    loop = jnp.arange(_N, dtype=edge_index.dtype)
    pad_ids = (jnp.arange(_MP - _E - _N, dtype=jnp.int32) * 7) % _N
    src_pad = jnp.concatenate([edge_index[0], loop, pad_ids])
    dst_safe = jnp.concatenate([edge_index[1], loop, pad_ids])
    dst_pad = jnp.concatenate([
        edge_index[1], loop,
        jnp.full((_MP - _E - _N,), _SENTINEL, edge_index.dtype),
    ])

    posp = jnp.pad(pos, ((0, 0), (0, 13)))
    gpd = _gather_sc(posp, dst_safe)
    t1 = jnp.pad(jnp.concatenate([x, pos], axis=1), ((0, 0), (0, 10)))
    x1 = _conv_layer(t1, gpd, src_pad, dst_pad, 3, params['ln1'], params['gn1'])
    t2 = jnp.pad(jnp.concatenate([x1, pos], axis=1), ((0, 0), (0, 13)))
    x2 = _conv_layer(t2, gpd, src_pad, dst_pad, 128, params['ln2'], params['gn2'])

    x_add = jax.ops.segment_sum(x2, batch, num_segments=_G)
    cnt = jax.ops.segment_sum(jnp.ones((_N, 1), jnp.float32), batch, num_segments=_G)
    x_mean = x_add / jnp.maximum(cnt, 1.0)
    x_max = jax.ops.segment_max(x2, batch, num_segments=_G)
    h = jnp.concatenate([x_max, x_mean, x_add], axis=1)
    wl, bl = params['lin1']
    return h @ wl + bl


# speedup vs baseline: 2.2055x; 1.0234x over previous
import functools

import jax
import jax.numpy as jnp
from jax import lax
from jax.experimental import pallas as pl
from jax.experimental.pallas import tpu as pltpu
from jax.experimental.pallas import tpu_sc as plsc

_N = 10000
_E = 320000
_G = 64
_NW = 32
_RPW = 320
_NPAD = _NW * _RPW
_SENTINEL = 1 << 29


def _elu(x):
    return jnp.where(x > 0, x, jnp.exp(jnp.minimum(x, 0.0)) - 1.0)


def _mlp3_body(x_ref, w1, b1, w2, b2, w3, b3, o_ref, *, elu_out):
    h = x_ref[...]
    h = _elu(jnp.dot(h, w1[...], preferred_element_type=jnp.float32) + b1[...])
    h = _elu(jnp.dot(h, w2[...], preferred_element_type=jnp.float32) + b2[...])
    h = jnp.dot(h, w3[...], preferred_element_type=jnp.float32) + b3[...]
    if elu_out:
        h = _elu(h)
    o_ref[...] = h


def _mlp3(x, params, elu_out=False, block=2048, pad_to=None, trim=True):
    (w1, b1), (w2, b2), (w3, b3) = params
    m, k = x.shape
    out_dim = w3.shape[1]
    mp = pad_to or ((m + block - 1) // block) * block
    assert mp % block == 0
    if mp != m:
        x = jnp.pad(x, ((0, mp - m), (0, 0)))
    grid = mp // block
    full = lambda r, c: pl.BlockSpec((r, c), lambda i: (0, 0))
    out = pl.pallas_call(
        functools.partial(_mlp3_body, elu_out=elu_out),
        grid=(grid,),
        in_specs=[
            pl.BlockSpec((block, k), lambda i: (i, 0)),
            full(*w1.shape), full(1, b1.shape[0]),
            full(*w2.shape), full(1, b2.shape[0]),
            full(*w3.shape), full(1, b3.shape[0]),
        ],
        out_specs=pl.BlockSpec((block, out_dim), lambda i: (i, 0)),
        out_shape=jax.ShapeDtypeStruct((mp, out_dim), jnp.float32),
    )(x, w1, b1.reshape(1, -1), w2, b2.reshape(1, -1), w3, b3.reshape(1, -1))
    return out[:m] if trim else out


def _mlp3_pair_body(xa_ref, xb_ref, w1a, w1b, b1, w2, b2, w3, b3, o_ref):
    h = (jnp.dot(xa_ref[...], w1a[...], preferred_element_type=jnp.float32)
         + jnp.dot(xb_ref[...], w1b[...], preferred_element_type=jnp.float32)
         + b1[...])
    h = _elu(h)
    h = _elu(jnp.dot(h, w2[...], preferred_element_type=jnp.float32) + b2[...])
    h = jnp.dot(h, w3[...], preferred_element_type=jnp.float32) + b3[...]
    half = h.shape[1] // 2
    lo = pltpu.bitcast(h[:, :half].astype(jnp.bfloat16), jnp.uint16).astype(jnp.uint32)
    hi = pltpu.bitcast(h[:, half:].astype(jnp.bfloat16), jnp.uint16).astype(jnp.uint32)
    o_ref[...] = pltpu.bitcast(lo | (hi << 16), jnp.int32)


def _mlp3_pair(xa, xb, w1a, w1b, b1, p2, p3, block=2048):
    (w2, b2), (w3, b3) = p2, p3
    m = xa.shape[0]
    out_dim = w3.shape[1]
    assert m % block == 0 and xb.shape[0] == m
    full = lambda r, c: pl.BlockSpec((r, c), lambda i: (0, 0))
    return pl.pallas_call(
        _mlp3_pair_body,
        grid=(m // block,),
        in_specs=[
            pl.BlockSpec((block, xa.shape[1]), lambda i: (i, 0)),
            pl.BlockSpec((block, xb.shape[1]), lambda i: (i, 0)),
            full(*w1a.shape), full(*w1b.shape), full(1, b1.shape[0]),
            full(*w2.shape), full(1, b2.shape[0]),
            full(*w3.shape), full(1, b3.shape[0]),
        ],
        out_specs=pl.BlockSpec((block, out_dim // 2), lambda i: (i, 0)),
        out_shape=jax.ShapeDtypeStruct((m, out_dim // 2), jnp.int32),
    )(xa, xb, w1a, w1b, b1.reshape(1, -1), w2, b2.reshape(1, -1),
      w3, b3.reshape(1, -1))


def _gather_sc(table, idx):
    v, d = table.shape
    mp = idx.shape[0]
    per_w = mp // _NW
    assert per_w * _NW == mp and (d * 4) % 64 == 0
    kb = 128 if d > 64 else 256
    nk = per_w // kb
    assert nk * kb == per_w
    mesh = plsc.VectorSubcoreMesh(core_axis_name="c", subcore_axis_name="s")

    @functools.partial(
        pl.kernel,
        mesh=mesh,
        compiler_params=pltpu.CompilerParams(
            needs_layout_passes=False, use_tc_tiling_on_sc=False),
        out_type=jax.ShapeDtypeStruct((mp, d), jnp.float32),
        scratch_types=[
            pltpu.VMEM((per_w,), jnp.int32),
            pltpu.VMEM((2 * kb, d), jnp.float32),
            pltpu.SemaphoreType.DMA,
        ],
    )
    def k(table_hbm, idx_hbm, out_hbm, idxv, buf, sem):
        wid = lax.axis_index("c") * 16 + lax.axis_index("s")
        wbase = wid * per_w
        pltpu.sync_copy(idx_hbm.at[pl.ds(wbase, per_w)], idxv)

        def _start(i):
            pltpu.make_async_copy(
                table_hbm.at[idxv.at[pl.ds(i * kb, kb)]],
                buf.at[pl.ds((i % 2) * kb, kb)], sem).start()

        def _drain():
            pltpu.make_async_copy(
                table_hbm.at[idxv.at[pl.ds(0, kb)]],
                buf.at[pl.ds(0, kb)], sem).wait()

        _start(jnp.int32(0))

        def body(i, _):
            _start(jnp.minimum(i + 1, nk - 1))
            _drain()
            pltpu.sync_copy(buf.at[pl.ds((i % 2) * kb, kb)],
                            out_hbm.at[pl.ds(wbase + i * kb, kb)])
            return 0

        lax.fori_loop(0, nk, body, 0)
        _drain()

    return k(table, idx)


def _segmax_sc(msg, dst):
    mp, d = msg.shape
    ch = 8192
    fb = 512
    rb = 256
    sb = 32
    assert mp % ch == 0 and d % 16 == 0
    n_chunks = mp // ch
    ncol = d // 16
    mesh = plsc.VectorSubcoreMesh(core_axis_name="c", subcore_axis_name="s")

    @functools.partial(
        pl.kernel,
        mesh=mesh,
        compiler_params=pltpu.CompilerParams(needs_layout_passes=False),
        out_type=jax.ShapeDtypeStruct((_NPAD, d), jnp.int32),
        scratch_types=[
            pltpu.VMEM((_RPW + 1, d), jnp.int32),
            pltpu.VMEM((ch,), jnp.int32),
            pltpu.VMEM((fb,), jnp.int32),
            pltpu.VMEM((fb,), jnp.int32),
            pltpu.VMEM((rb, d), jnp.int32),
            pltpu.SemaphoreType.DMA,
        ],
    )
    def k(msg_hbm, dst_hbm, out_hbm, acc, dstv, fid, rid, rows, sem):
        wid = lax.axis_index("c") * 16 + lax.axis_index("s")
        base = wid * _RPW
        lanes = lax.iota(jnp.int32, 16)
        neg = jnp.full((16,), -8323200, jnp.int32)

        def init_row(r, _):
            for kk in range(ncol):
                acc[r, pl.ds(kk * 16, 16)] = neg
            return 0
        lax.fori_loop(0, _RPW + 1, init_row, 0)

        def reset_bufs():
            for t in range(fb // 16):
                fid[pl.ds(t * 16, 16)] = t * 16 + lanes
                rid[pl.ds(t * 16, 16)] = jnp.full((16,), _RPW, jnp.int32)
        reset_bufs()

        nslot = rb // sb

        def _start(i):
            pltpu.make_async_copy(
                msg_hbm.at[fid.at[pl.ds(i * sb, sb)]],
                rows.at[pl.ds((i % nslot) * sb, sb)], sem).start()

        def _drain():
            pltpu.make_async_copy(
                msg_hbm.at[fid.at[pl.ds(0, sb)]],
                rows.at[pl.ds(0, sb)], sem).wait()

        def flush(cnt):
            nsb_d = jnp.maximum((cnt + sb - 1) // sb, 1)
            _start(jnp.int32(0))

            def body(i, _):
                _drain()
                _start(jnp.minimum(i + 1, nsb_d - 1))
                slot = (i % nslot) * sb

                def acc_row(j16, _):
                    rv = rid[pl.ds(i * sb + j16 * 16, 16)]
                    for l in range(16):
                        r = rv[l]
                        j = slot + j16 * 16 + l
                        for kk in range(ncol):
                            sl = pl.ds(kk * 16, 16)
                            a = plsc.bitcast(acc[r, sl], jnp.bfloat16)
                            b = plsc.bitcast(rows[j, sl], jnp.bfloat16)
                            acc[r, sl] = plsc.bitcast(jnp.maximum(a, b), jnp.int32)
                    return 0
                lax.fori_loop(0, sb // 16, acc_row, 0)
                return 0

            lax.fori_loop(0, nsb_d, body, 0)
            _drain()
            reset_bufs()
            return jnp.int32(0)

        def chunk(c, cnt):
            pltpu.sync_copy(dst_hbm.at[pl.ds(c * ch, ch)], dstv)

            def block8(b, cnt):
                cnt = lax.cond(cnt > fb - 128, flush, lambda x: x, cnt)
                for g8 in range(8):
                    g = b * 8 + g8
                    v = dstv[pl.ds(g * 16, 16)]
                    rel = v - base
                    m = plsc.bitcast(rel, jnp.uint32) < jnp.uint32(_RPW)
                    eid = c * ch + g * 16 + lanes
                    plsc.store_compressed(fid.at[pl.ds(cnt, 16)], eid, mask=m)
                    plsc.store_compressed(rid.at[pl.ds(cnt, 16)], rel, mask=m)
                    cnt = cnt + plsc.all_reduce_population_count(m)[0]
                return cnt

            return lax.fori_loop(0, ch // 128, block8, cnt)

        cnt = lax.fori_loop(0, n_chunks, chunk, jnp.int32(0))
        flush(cnt)
        pltpu.sync_copy(acc.at[pl.ds(0, _RPW)], out_hbm.at[pl.ds(base, _RPW)])

    return k(msg, dst)


_MP = 335872


def _split_w1(w1, nf, da):
    wf, wp = w1[:nf], w1[nf:]
    w1a = jnp.zeros((da, w1.shape[1]), jnp.float32)
    w1a = w1a.at[:nf].set(wf).at[nf:nf + 3].set(wp)
    w1b = jnp.zeros((16, w1.shape[1]), jnp.float32).at[:3].set(-wp)
    return w1a, w1b


def _conv_layer(feat_pos_tab, gpd, src_pad, dst_pad, nf, local_p, global_p):
    (w1, b1), p2, p3 = local_p
    gsrc = _gather_sc(feat_pos_tab, src_pad)
    w1a, w1b = _split_w1(w1, nf, feat_pos_tab.shape[1])
    msg = _mlp3_pair(gsrc, gpd, w1a, w1b, b1, p2, p3)
    agg32 = jnp.uint32(_segmax_sc(msg, dst_pad)[:_N])
    lo = lax.bitcast_convert_type((agg32 & 0xFFFF).astype(jnp.uint16), jnp.bfloat16)
    hi = lax.bitcast_convert_type((agg32 >> 16).astype(jnp.uint16), jnp.bfloat16)
    agg = jnp.concatenate([lo, hi], axis=1).astype(jnp.float32)
    return _mlp3(agg, global_p, elu_out=True)


def _pool_body(x_ref, b_ref, wl1, wl2, wl3, bl, o_ref, maxs, sums, cnts):
    i = pl.program_id(0)

    @pl.when(i == 0)
    def _():
        maxs[...] = jnp.full_like(maxs[...], -jnp.inf)
        sums[...] = jnp.zeros_like(sums[...])
        cnts[...] = jnp.zeros_like(cnts[...])

    xb = x_ref[...]
    bb = b_ref[...]
    onehot = (bb == lax.broadcasted_iota(jnp.int32, (1, _G), 1)).astype(jnp.float32)
    dn = (((0,), (0,)), ((), ()))
    sums[...] += lax.dot_general(onehot, xb, dn, preferred_element_type=jnp.float32)
    cnts[...] += lax.dot_general(onehot, jnp.ones_like(xb), dn,
                                 preferred_element_type=jnp.float32)
    for g in range(_G):
        masked = jnp.where(bb == g, xb, -jnp.inf)
        maxs[g, :] = jnp.maximum(maxs[g, :], jnp.max(masked, axis=0))

    @pl.when(i == pl.num_programs(0) - 1)
    def _():
        mean = sums[...] / jnp.maximum(cnts[...], 1.0)
        o_ref[...] = (
            jnp.dot(maxs[...], wl1[...], preferred_element_type=jnp.float32)
            + jnp.dot(mean, wl2[...], preferred_element_type=jnp.float32)
            + jnp.dot(sums[...], wl3[...], preferred_element_type=jnp.float32)
            + bl[...])


def _pool_tc(x2, batch, wl, bl, block=2048):
    n, dcol = x2.shape
    npad = ((n + block - 1) // block) * block
    x2p = jnp.pad(x2, ((0, npad - n), (0, 0)))
    bp = jnp.pad(batch.reshape(-1, 1), ((0, npad - n), (0, 0)),
                 constant_values=_G)
    wlp = jnp.pad(wl, ((0, 0), (0, 128 - wl.shape[1])))
    blp = jnp.pad(bl.reshape(1, -1), ((0, 0), (0, 128 - bl.shape[0])))
    full = lambda r, c: pl.BlockSpec((r, c), lambda i: (0, 0))
    out = pl.pallas_call(
        _pool_body,
        grid=(npad // block,),
        in_specs=[
            pl.BlockSpec((block, dcol), lambda i: (i, 0)),
            pl.BlockSpec((block, 1), lambda i: (i, 0)),
            full(dcol, 128), full(dcol, 128), full(dcol, 128), full(1, 128),
        ],
        out_specs=full(_G, 128),
        out_shape=jax.ShapeDtypeStruct((_G, 128), jnp.float32),
        scratch_shapes=[pltpu.VMEM((_G, dcol), jnp.float32)] * 3,
    )(x2p, bp, wlp[:dcol], wlp[dcol:2 * dcol], wlp[2 * dcol:], blp)
    return out[:, : wl.shape[1]]


def kernel(x, pos, params, edge_index, batch):
    loop = jnp.arange(_N, dtype=edge_index.dtype)
    pad_ids = (jnp.arange(_MP - _E - _N, dtype=jnp.int32) * 7) % _N
    src_pad = jnp.concatenate([edge_index[0], loop, pad_ids])
    dst_safe = jnp.concatenate([edge_index[1], loop, pad_ids])
    dst_pad = jnp.concatenate([
        edge_index[1], loop,
        jnp.full((_MP - _E - _N,), _SENTINEL, edge_index.dtype),
    ])

    posp = jnp.pad(pos, ((0, 0), (0, 13)))
    gpd = _gather_sc(posp, dst_safe)
    t1 = jnp.pad(jnp.concatenate([x, pos], axis=1), ((0, 0), (0, 10)))
    x1 = _conv_layer(t1, gpd, src_pad, dst_pad, 3, params['ln1'], params['gn1'])
    t2 = jnp.pad(jnp.concatenate([x1, pos], axis=1), ((0, 0), (0, 13)))
    x2 = _conv_layer(t2, gpd, src_pad, dst_pad, 128, params['ln2'], params['gn2'])

    wl, bl = params['lin1']
    return _pool_tc(x2, batch, wl, bl)
